# Initial kernel scaffold; baseline (speedup 1.0000x reference)
import jax
import jax.numpy as jnp
from jax.experimental import pallas as pl


def kernel(x, y, x0, y0, image, pixelscale, scale):
    raise NotImplementedError("write your pallas kernel here")


# speedup vs baseline: 50.3178x; 50.3178x over previous
import functools

import jax
import jax.numpy as jnp
from jax import lax
from jax.experimental import pallas as pl
from jax.experimental.pallas import tpu as pltpu
from jax.experimental.pallas import tpu_sc as plsc

L = 16
CHUNK = 1024
GROUP = 128


@functools.lru_cache(maxsize=None)
def _build(n, h, w):
    info = plsc.get_sparse_core_info()
    nc, ns = info.num_cores, info.num_subcores
    nw = nc * ns
    assert n % (nw * CHUNK) == 0
    nq = n // nw
    nchunk = nq // CHUNK
    vecs = CHUNK // L
    groups = CHUNK // GROUP
    wf = float(w)
    hf = float(h)
    tmax = 4 * ((h - 2) * w + (w - 2)) + 4

    mesh = plsc.VectorSubcoreMesh(core_axis_name="c", subcore_axis_name="s")

    @functools.partial(
        pl.kernel,
        out_type=jax.ShapeDtypeStruct((n,), jnp.float32),
        mesh=mesh,
        scratch_types=[
            pltpu.VMEM((CHUNK,), jnp.float32),
            pltpu.VMEM((CHUNK,), jnp.float32),
            pltpu.VMEM((CHUNK,), jnp.int32),
            pltpu.VMEM((CHUNK,), jnp.int32),
            pltpu.VMEM((CHUNK,), jnp.int32),
            pltpu.VMEM((CHUNK,), jnp.int32),
            pltpu.VMEM((CHUNK,), jnp.float32),
            pltpu.VMEM((CHUNK,), jnp.float32),
            pltpu.VMEM((CHUNK,), jnp.float32),
            pltpu.VMEM((CHUNK,), jnp.float32),
            pltpu.VMEM((CHUNK,), jnp.float32),
            pltpu.VMEM((CHUNK,), jnp.float32),
            pltpu.VMEM((CHUNK,), jnp.float32),
            pltpu.VMEM((CHUNK,), jnp.float32),
            pltpu.VMEM((5 * L,), jnp.float32),
            pltpu.SemaphoreType.DMA,
        ],
    )
    def run(xf, yf, table, params, out, xv, yv, idxa, idxc, idxb, idxd,
            wdx0, wdy0, wdy1, fav, fcv, fbv, fdv, outv, pv, gsem):
        cid = lax.axis_index("c")
        sid = lax.axis_index("s")
        wid = sid * nc + cid
        pltpu.sync_copy(params, pv)
        x0v = pv[pl.ds(0 * L, L)]
        y0v = pv[pl.ds(1 * L, L)]
        sxv = pv[pl.ds(2 * L, L)]
        syv = pv[pl.ds(3 * L, L)]
        sclv = pv[pl.ds(4 * L, L)]
        base0 = wid * nq

        @pl.loop(0, nchunk)
        def _chunk(t):
            base = base0 + t * CHUNK
            pltpu.sync_copy(xf.at[pl.ds(base, CHUNK)], xv)
            pltpu.sync_copy(yf.at[pl.ds(base, CHUNK)], yv)

            @pl.loop(0, vecs)
            def _pass_a(i):
                off = i * L
                xq = xv[pl.ds(off, L)]
                yq = yv[pl.ds(off, L)]
                xn = (xq - x0v) * sxv
                yn = (yq - y0v) * syv
                oob = (yn < -1.0) | (yn > 1.0) | (xn < -1.0) | (xn > 1.0)
                xp = 0.5 * ((xn + 1.0) * wf - 1.0)
                yp = 0.5 * ((yn + 1.0) * hf - 1.0)
                xpc = jnp.minimum(jnp.maximum(xp, 0.0), wf - 2.0)
                ypc = jnp.minimum(jnp.maximum(yp, 0.0), hf - 2.0)
                x0i = xpc.astype(jnp.int32)
                y0i = ypc.astype(jnp.int32)
                dx0 = xp - x0i.astype(jnp.float32)
                dy0 = yp - y0i.astype(jnp.float32)
                dy1 = 1.0 - dy0
                zero = jnp.zeros((L,), jnp.float32)
                ib = (y0i * w + x0i) * 4
                idxa[pl.ds(off, L)] = ib
                idxc[pl.ds(off, L)] = ib + 1
                idxb[pl.ds(off, L)] = ib + 2
                idxd[pl.ds(off, L)] = ib + 3
                wdx0[pl.ds(off, L)] = dx0
                wdy0[pl.ds(off, L)] = jnp.where(oob, zero, dy0)
                wdy1[pl.ds(off, L)] = jnp.where(oob, zero, dy1)

            copies = []
            for g in range(groups):
                sl = pl.ds(g * GROUP, GROUP)
                copies += [
                    pltpu.async_copy(table.at[ix.at[sl]], buf.at[sl], gsem)
                    for ix, buf in ((idxa, fav), (idxc, fcv),
                                    (idxb, fbv), (idxd, fdv))
                ]
            for cp in copies:
                cp.wait()

            @pl.loop(0, vecs)
            def _pass_b(i):
                off = i * L
                fa = fav[pl.ds(off, L)] * sclv
                fc = fcv[pl.ds(off, L)] * sclv
                fb = fbv[pl.ds(off, L)] * sclv
                fd = fdv[pl.ds(off, L)] * sclv
                dx0 = wdx0[pl.ds(off, L)]
                dy0 = wdy0[pl.ds(off, L)]
                dy1 = wdy1[pl.ds(off, L)]
                dx1 = 1.0 - dx0
                t1 = fa * dx1 + fc * dx0
                t0 = fb * dx1 + fd * dx0
                outv[pl.ds(off, L)] = dy1 * t1 + dy0 * t0

            pltpu.sync_copy(outv, out.at[pl.ds(base, CHUNK)])

    return run


def kernel(x, y, x0, y0, image, pixelscale, scale):
    h, w = image.shape
    n = x.size
    xf = x.reshape(-1)
    yf = y.reshape(-1)
    flat = image.reshape(-1)
    right = jnp.concatenate([flat[1:], flat[:1]])
    down = jnp.concatenate([flat[w:], flat[:w]])
    downright = jnp.concatenate([flat[w + 1:], flat[:w + 1]])
    table = jnp.stack([flat, right, down, downright], axis=1).reshape(-1)
    fov_x = pixelscale * w
    fov_y = pixelscale * h
    sx = jnp.float32(2.0) / fov_x
    sy = jnp.float32(2.0) / fov_y
    params = jnp.concatenate(
        [jnp.full((L,), v, jnp.float32) for v in (x0, y0, sx, sy, scale)])
    out = _build(n, h, w)(xf, yf, table, params)
    return out.reshape(x.shape)


# speedup vs baseline: 328.7578x; 6.5336x over previous
import functools

import jax
import jax.numpy as jnp
from jax import lax
from jax.experimental import pallas as pl
from jax.experimental.pallas import tpu as pltpu
from jax.experimental.pallas import tpu_sc as plsc

L = 16
CHUNK = 1024
GROUP = 128


@functools.lru_cache(maxsize=None)
def _build(n, h, w):
    info = plsc.get_sparse_core_info()
    nc, ns = info.num_cores, info.num_subcores
    nw = nc * ns
    assert n % (nw * CHUNK) == 0
    nq = n // nw
    nchunk = nq // CHUNK
    vecs = CHUNK // L
    groups = CHUNK // GROUP
    seg = (h * w) // ns
    wf = float(w)
    hf = float(h)

    mesh = plsc.VectorSubcoreMesh(core_axis_name="c", subcore_axis_name="s")

    @functools.partial(
        pl.kernel,
        out_type=jax.ShapeDtypeStruct((n,), jnp.float32),
        mesh=mesh,
        scratch_types=[
            pltpu.VMEM_SHARED((h * w,), jnp.float32),
            pltpu.VMEM((CHUNK,), jnp.float32),
            pltpu.VMEM((CHUNK,), jnp.float32),
            pltpu.VMEM((CHUNK,), jnp.int32),
            pltpu.VMEM((CHUNK,), jnp.int32),
            pltpu.VMEM((CHUNK,), jnp.int32),
            pltpu.VMEM((CHUNK,), jnp.int32),
            pltpu.VMEM((CHUNK,), jnp.float32),
            pltpu.VMEM((CHUNK,), jnp.float32),
            pltpu.VMEM((CHUNK,), jnp.float32),
            pltpu.VMEM((CHUNK,), jnp.float32),
            pltpu.VMEM((CHUNK,), jnp.float32),
            pltpu.VMEM((CHUNK,), jnp.float32),
            pltpu.VMEM((CHUNK,), jnp.float32),
            pltpu.VMEM((CHUNK,), jnp.float32),
            pltpu.VMEM((5 * L,), jnp.float32),
            pltpu.SemaphoreType.DMA,
        ],
    )
    def run(xf, yf, img, params, out, shared, xv, yv, idxa, idxc, idxb, idxd,
            wdx0, wdy0, wdy1, fav, fcv, fbv, fdv, outv, pv, gsem):
        cid = lax.axis_index("c")
        sid = lax.axis_index("s")
        wid = sid * nc + cid
        sbase = sid * seg
        pltpu.sync_copy(img.at[pl.ds(sbase, seg)], shared.at[pl.ds(sbase, seg)])
        pltpu.sync_copy(params, pv)
        plsc.subcore_barrier()
        x0v = pv[pl.ds(0 * L, L)]
        y0v = pv[pl.ds(1 * L, L)]
        sxv = pv[pl.ds(2 * L, L)]
        syv = pv[pl.ds(3 * L, L)]
        sclv = pv[pl.ds(4 * L, L)]
        base0 = wid * nq

        @pl.loop(0, nchunk)
        def _chunk(t):
            base = base0 + t * CHUNK
            pltpu.sync_copy(xf.at[pl.ds(base, CHUNK)], xv)
            pltpu.sync_copy(yf.at[pl.ds(base, CHUNK)], yv)

            @pl.loop(0, vecs)
            def _pass_a(i):
                off = i * L
                xq = xv[pl.ds(off, L)]
                yq = yv[pl.ds(off, L)]
                xn = (xq - x0v) * sxv
                yn = (yq - y0v) * syv
                oob = (yn < -1.0) | (yn > 1.0) | (xn < -1.0) | (xn > 1.0)
                xp = 0.5 * ((xn + 1.0) * wf - 1.0)
                yp = 0.5 * ((yn + 1.0) * hf - 1.0)
                xpc = jnp.minimum(jnp.maximum(xp, 0.0), wf - 2.0)
                ypc = jnp.minimum(jnp.maximum(yp, 0.0), hf - 2.0)
                x0i = xpc.astype(jnp.int32)
                y0i = ypc.astype(jnp.int32)
                dx0 = xp - x0i.astype(jnp.float32)
                dy0 = yp - y0i.astype(jnp.float32)
                dy1 = 1.0 - dy0
                zero = jnp.zeros((L,), jnp.float32)
                ia = y0i * w + x0i
                idxa[pl.ds(off, L)] = ia
                idxc[pl.ds(off, L)] = ia + 1
                idxb[pl.ds(off, L)] = ia + w
                idxd[pl.ds(off, L)] = ia + (w + 1)
                wdx0[pl.ds(off, L)] = dx0
                wdy0[pl.ds(off, L)] = jnp.where(oob, zero, dy0)
                wdy1[pl.ds(off, L)] = jnp.where(oob, zero, dy1)

            copies = []
            for g in range(groups):
                sl = pl.ds(g * GROUP, GROUP)
                copies += [
                    pltpu.async_copy(shared.at[ix.at[sl]], buf.at[sl], gsem)
                    for ix, buf in ((idxa, fav), (idxc, fcv),
                                    (idxb, fbv), (idxd, fdv))
                ]
            for cp in copies:
                cp.wait()

            @pl.loop(0, vecs)
            def _pass_b(i):
                off = i * L
                fa = fav[pl.ds(off, L)] * sclv
                fc = fcv[pl.ds(off, L)] * sclv
                fb = fbv[pl.ds(off, L)] * sclv
                fd = fdv[pl.ds(off, L)] * sclv
                dx0 = wdx0[pl.ds(off, L)]
                dy0 = wdy0[pl.ds(off, L)]
                dy1 = wdy1[pl.ds(off, L)]
                dx1 = 1.0 - dx0
                t1 = fa * dx1 + fc * dx0
                t0 = fb * dx1 + fd * dx0
                outv[pl.ds(off, L)] = dy1 * t1 + dy0 * t0

            pltpu.sync_copy(outv, out.at[pl.ds(base, CHUNK)])

    return run


def kernel(x, y, x0, y0, image, pixelscale, scale):
    h, w = image.shape
    n = x.size
    xf = x.reshape(-1)
    yf = y.reshape(-1)
    img = image.reshape(-1)
    fov_x = pixelscale * w
    fov_y = pixelscale * h
    sx = jnp.float32(2.0) / fov_x
    sy = jnp.float32(2.0) / fov_y
    params = jnp.concatenate(
        [jnp.full((L,), v, jnp.float32) for v in (x0, y0, sx, sy, scale)])
    out = _build(n, h, w)(xf, yf, img, params)
    return out.reshape(x.shape)


# speedup vs baseline: 343.8369x; 1.0459x over previous
import functools

import jax
import jax.numpy as jnp
from jax import lax
from jax.experimental import pallas as pl
from jax.experimental.pallas import tpu as pltpu
from jax.experimental.pallas import tpu_sc as plsc

L = 16
CHUNK = 1024
GROUP = 128


@functools.lru_cache(maxsize=None)
def _build(n, h, w):
    info = plsc.get_sparse_core_info()
    nc, ns = info.num_cores, info.num_subcores
    nw = nc * ns
    assert n % (nw * 2 * CHUNK) == 0
    nq = n // nw
    nchunk = nq // CHUNK
    npair = nchunk // 2
    groups = CHUNK // GROUP
    gvecs = GROUP // L
    seg = (h * w) // ns
    wf = float(w)
    hf = float(h)

    mesh = plsc.VectorSubcoreMesh(core_axis_name="c", subcore_axis_name="s")

    @functools.partial(
        pl.kernel,
        out_type=jax.ShapeDtypeStruct((n,), jnp.float32),
        mesh=mesh,
        scratch_types=[
            pltpu.VMEM_SHARED((h * w,), jnp.float32),
            [pltpu.VMEM((CHUNK,), jnp.float32)] * 2,
            [pltpu.VMEM((CHUNK,), jnp.float32)] * 2,
            pltpu.VMEM((CHUNK,), jnp.int32),
            pltpu.VMEM((CHUNK,), jnp.int32),
            pltpu.VMEM((CHUNK,), jnp.int32),
            pltpu.VMEM((CHUNK,), jnp.int32),
            pltpu.VMEM((CHUNK,), jnp.float32),
            pltpu.VMEM((CHUNK,), jnp.float32),
            pltpu.VMEM((CHUNK,), jnp.float32),
            pltpu.VMEM((CHUNK,), jnp.float32),
            pltpu.VMEM((CHUNK,), jnp.float32),
            pltpu.VMEM((CHUNK,), jnp.float32),
            pltpu.VMEM((CHUNK,), jnp.float32),
            pltpu.VMEM((CHUNK,), jnp.float32),
            pltpu.VMEM((5 * L,), jnp.float32),
            [pltpu.SemaphoreType.DMA] * 8,
            [pltpu.SemaphoreType.DMA] * 2,
        ],
    )
    def run(xf, yf, img, params, out, shared, xvs, yvs, idxa, idxc, idxb,
            idxd, wdx0, wdy0, wdy1, fav, fcv, fbv, fdv, outv, pv, gsems,
            xysems):
        cid = lax.axis_index("c")
        sid = lax.axis_index("s")
        wid = sid * nc + cid
        sbase = sid * seg
        pltpu.sync_copy(img.at[pl.ds(sbase, seg)], shared.at[pl.ds(sbase, seg)])
        pltpu.sync_copy(params, pv)
        plsc.subcore_barrier()
        x0v = pv[pl.ds(0 * L, L)]
        y0v = pv[pl.ds(1 * L, L)]
        sxv = pv[pl.ds(2 * L, L)]
        syv = pv[pl.ds(3 * L, L)]
        sclv = pv[pl.ds(4 * L, L)]
        base0 = wid * nq

        def fire_xy(t, s):
            b = base0 + t * CHUNK
            pltpu.async_copy(xf.at[pl.ds(b, CHUNK)], xvs[s], xysems[s])
            pltpu.async_copy(yf.at[pl.ds(b, CHUNK)], yvs[s], xysems[s])

        def wait_xy(t, s):
            b = base0 + t * CHUNK
            for src, dst in ((xf, xvs[s]), (yf, yvs[s])):
                pltpu.make_async_copy(
                    src.at[pl.ds(b, CHUNK)], dst, xysems[s]).wait()

        def process(t, s):
            base = base0 + t * CHUNK
            xv = xvs[s]
            yv = yvs[s]
            fired = []
            for g in range(groups):
                goff = g * GROUP

                @pl.loop(0, gvecs)
                def _pass_a(i):
                    off = goff + i * L
                    sl = pl.ds(off, L)
                    xq = xv[sl]
                    yq = yv[sl]
                    xn = (xq - x0v) * sxv
                    yn = (yq - y0v) * syv
                    oob = (yn < -1.0) | (yn > 1.0) | (xn < -1.0) | (xn > 1.0)
                    xp = 0.5 * ((xn + 1.0) * wf - 1.0)
                    yp = 0.5 * ((yn + 1.0) * hf - 1.0)
                    xpc = jnp.minimum(jnp.maximum(xp, 0.0), wf - 2.0)
                    ypc = jnp.minimum(jnp.maximum(yp, 0.0), hf - 2.0)
                    x0i = xpc.astype(jnp.int32)
                    y0i = ypc.astype(jnp.int32)
                    dx0 = xp - x0i.astype(jnp.float32)
                    dy0 = yp - y0i.astype(jnp.float32)
                    dy1 = 1.0 - dy0
                    zero = jnp.zeros((L,), jnp.float32)
                    ia = y0i * w + x0i
                    idxa[sl] = ia
                    idxc[sl] = ia + 1
                    idxb[sl] = ia + w
                    idxd[sl] = ia + (w + 1)
                    wdx0[sl] = dx0
                    wdy0[sl] = jnp.where(oob, zero, dy0)
                    wdy1[sl] = jnp.where(oob, zero, dy1)

                gsl = pl.ds(goff, GROUP)
                fired.append([
                    pltpu.async_copy(shared.at[ix.at[gsl]], buf.at[gsl],
                                     gsems[g])
                    for ix, buf in ((idxa, fav), (idxc, fcv),
                                    (idxb, fbv), (idxd, fdv))
                ])

            for g in range(groups):
                goff = g * GROUP
                for cp in fired[g]:
                    cp.wait()

                @pl.loop(0, gvecs)
                def _pass_b(i):
                    off = goff + i * L
                    sl = pl.ds(off, L)
                    fa = fav[sl] * sclv
                    fc = fcv[sl] * sclv
                    fb = fbv[sl] * sclv
                    fd = fdv[sl] * sclv
                    dx0 = wdx0[sl]
                    dy0 = wdy0[sl]
                    dy1 = wdy1[sl]
                    dx1 = 1.0 - dx0
                    t1 = fa * dx1 + fc * dx0
                    t0 = fb * dx1 + fd * dx0
                    outv[sl] = dy1 * t1 + dy0 * t0

            pltpu.sync_copy(outv, out.at[pl.ds(base, CHUNK)])

        fire_xy(0, 0)

        @pl.loop(0, npair)
        def _pair(u):
            t0 = u * 2
            t1 = t0 + 1
            wait_xy(t0, 0)
            fire_xy(t1, 1)
            process(t0, 0)
            wait_xy(t1, 1)

            @pl.when(u + 1 < npair)
            def _pf():
                fire_xy(t1 + 1, 0)

            process(t1, 1)

    return run


def kernel(x, y, x0, y0, image, pixelscale, scale):
    h, w = image.shape
    n = x.size
    xf = x.reshape(-1)
    yf = y.reshape(-1)
    img = image.reshape(-1)
    fov_x = pixelscale * w
    fov_y = pixelscale * h
    sx = jnp.float32(2.0) / fov_x
    sy = jnp.float32(2.0) / fov_y
    params = jnp.concatenate(
        [jnp.full((L,), v, jnp.float32) for v in (x0, y0, sx, sy, scale)])
    out = _build(n, h, w)(xf, yf, img, params)
    return out.reshape(x.shape)
